# baseline (speedup 1.0000x reference)
import functools

import jax
import jax.numpy as jnp
from jax import lax
from jax.experimental import pallas as pl
from jax.experimental.pallas import tpu as pltpu
from jax.experimental.pallas import tpu_sc as plsc

NC = 2
NS = 16
NW = NC * NS
LANES = 16

BCH = 8
CHUNK = 256
SUB = 128
KSUB = CHUNK // SUB


def _prep_body(p_ref, q_ref, sig_ref, al_ref, x_ref, stab_ref, dtab_ref):
    p = p_ref[...]
    q = q_ref[...]
    sig = sig_ref[...]
    inv = 1.0 / (sig * sig)
    xa = al_ref[...] * x_ref[...].T
    pn = jnp.sum(p * p, axis=1, keepdims=True)
    qn = jnp.sum(q * q, axis=1, keepdims=True)
    z1 = jnp.zeros_like(sig)
    z3 = jnp.zeros_like(p)
    stab_ref[...] = jnp.concatenate([2.0 * inv * p, -pn * inv, -inv, z3, xa],
                                    axis=1)
    dtab_ref[...] = jnp.concatenate(
        [q, z1, qn, jnp.zeros((q.shape[0], 11), jnp.float32)], axis=1)


def _edge_body(nchunk, rows_per_w, acc_per_tile,
               stab, dtab, sidx_hbm, didx_hbm, zeros_hbm, out_hbm,
               sidx_all, didx_all,
               srows0, drows0, contrib0,
               srows1, drows1, contrib1,
               acc, gsem0, gsem1, ssem0, ssem1):
    c = lax.axis_index("c")
    s = lax.axis_index("s")
    wid = s * NC + c

    pltpu.sync_copy(zeros_hbm.at[pl.ds(s * acc_per_tile, acc_per_tile)],
                    acc.at[pl.ds(s * acc_per_tile, acc_per_tile)])
    base = wid * rows_per_w
    plsc.subcore_barrier()

    bufs = ((srows0, drows0, contrib0, gsem0, ssem0),
            (srows1, drows1, contrib1, gsem1, ssem1))

    def fire_gathers(k, b):
        srows, drows, gsem = b[0], b[1], b[3]
        for j in range(KSUB):
            pltpu.async_copy(stab.at[sidx_all.at[k * KSUB + j]],
                             srows.at[pl.ds(j * SUB, SUB)], gsem)
            pltpu.async_copy(dtab.at[didx_all.at[k * KSUB + j]],
                             drows.at[pl.ds(j * SUB, SUB)], gsem)

    def drain_gathers(k, b):
        srows, drows, gsem = b[0], b[1], b[3]
        for j in range(KSUB):
            pltpu.make_async_copy(stab.at[sidx_all.at[k * KSUB + j]],
                                  srows.at[pl.ds(j * SUB, SUB)], gsem).wait()
            pltpu.make_async_copy(dtab.at[didx_all.at[k * KSUB + j]],
                                  drows.at[pl.ds(j * SUB, SUB)], gsem).wait()

    def compute(b):
        srows, drows, contrib = b[0], b[1], b[2]

        unroll = 4

        def cbody(i, _):
            prods = []
            for u in range(unroll):
                rows = lax.iota(jnp.int32, LANES) + (i * unroll + u) * LANES

                def gs(col, rows=rows):
                    return plsc.load_gather(
                        srows, [rows, jnp.full((LANES,), col, jnp.int32)])

                def gd(col, rows=rows):
                    return plsc.load_gather(
                        drows, [rows, jnp.full((LANES,), col, jnp.int32)])

                y = ((gs(0) * gd(0) + gs(1) * gd(1))
                     + (gs(2) * gd(2) + gs(3))
                     + gs(4) * gd(4))
                w = jnp.exp(y)
                prods.append((rows, [w * gs(8 + b_) for b_ in range(BCH)]))
            for rows, vals in prods:
                for b_ in range(BCH):
                    plsc.store_scatter(
                        contrib, [rows, jnp.full((LANES,), b_, jnp.int32)],
                        vals[b_])
            return 0

        lax.fori_loop(0, CHUNK // (LANES * unroll), cbody, 0)

    def fire_scatters(k, b):
        contrib, ssem = b[2], b[4]
        for j in range(KSUB):
            pltpu.async_copy(contrib.at[pl.ds(j * SUB, SUB)],
                             acc.at[didx_all.at[k * KSUB + j]], ssem, add=True)

    def drain_scatters(k, b):
        contrib, ssem = b[2], b[4]
        for j in range(KSUB):
            pltpu.make_async_copy(contrib.at[pl.ds(j * SUB, SUB)],
                                  acc.at[didx_all.at[k * KSUB + j]],
                                  ssem).wait()

    nchunk_h = nchunk // 2
    rows_h = rows_per_w // 2

    def phase(t, k, this_b, other_b):
        @pl.when(k + 1 < nchunk_h)
        def _():
            fire_gathers(k + 1, other_b)
        drain_gathers(k, this_b)

        @pl.when(t >= 1)
        def _():
            drain_scatters(k, this_b)
        compute(this_b)
        fire_scatters(k, this_b)

    for h in range(2):
        pltpu.sync_copy(sidx_hbm.at[pl.ds(base + h * rows_h, rows_h)],
                        sidx_all)
        pltpu.sync_copy(didx_hbm.at[pl.ds(base + h * rows_h, rows_h)],
                        didx_all)
        fire_gathers(0, bufs[0])

        def body(t, carry):
            phase(t, 2 * t, bufs[0], bufs[1])
            phase(t, 2 * t + 1, bufs[1], bufs[0])
            return carry

        lax.fori_loop(0, nchunk_h // 2, body, 0)
        drain_scatters(0, bufs[0])
        drain_scatters(0, bufs[1])

    plsc.subcore_barrier()
    pltpu.sync_copy(acc.at[pl.ds(s * acc_per_tile, acc_per_tile)],
                    out_hbm.at[c].at[pl.ds(s * acc_per_tile, acc_per_tile)])


def _merge_body(part_ref, out_ref):
    sblk = part_ref[0] + part_ref[1]
    out_ref[...] = sblk.T


def kernel(x, inp_positions, out_positions, alpha, sigma, edge_list):
    n_in = inp_positions.shape[0]
    n_out = out_positions.shape[0]
    e = edge_list.shape[1]
    assert n_out == n_in

    n_pad = -(-n_in // 128) * 128
    padn = n_pad - n_in
    p_pad = jnp.concatenate(
        [inp_positions, jnp.zeros((padn, 3), jnp.float32)], axis=0)
    q_pad = jnp.concatenate(
        [out_positions, jnp.zeros((padn, 3), jnp.float32)], axis=0)
    sig_pad = jnp.concatenate(
        [sigma.reshape(n_in, 1), jnp.ones((padn, 1), jnp.float32)], axis=0)
    al_pad = jnp.concatenate(
        [alpha.reshape(n_in, 1), jnp.zeros((padn, 1), jnp.float32)], axis=0)
    x_pad = jnp.concatenate(
        [x, jnp.zeros((BCH, padn), jnp.float32)], axis=1)

    cn = 2176
    assert n_pad % cn == 0
    grid = n_pad // cn
    stab, dtab = pl.pallas_call(
        _prep_body,
        grid=(grid,),
        in_specs=[
            pl.BlockSpec((cn, 3), lambda i: (i, 0)),
            pl.BlockSpec((cn, 3), lambda i: (i, 0)),
            pl.BlockSpec((cn, 1), lambda i: (i, 0)),
            pl.BlockSpec((cn, 1), lambda i: (i, 0)),
            pl.BlockSpec((BCH, cn), lambda i: (0, i)),
        ],
        out_specs=[
            pl.BlockSpec((cn, 16), lambda i: (i, 0)),
            pl.BlockSpec((cn, 16), lambda i: (i, 0)),
        ],
        out_shape=[
            jax.ShapeDtypeStruct((n_pad, 16), jnp.float32),
            jax.ShapeDtypeStruct((n_pad, 16), jnp.float32),
        ],
    )(p_pad, q_pad, sig_pad, al_pad, x_pad)

    ew = e // NW
    assert ew * NW == e and ew % 8 == 0
    nchunk = -(-ew // CHUNK)
    nchunk = -(-nchunk // 4) * 4
    per_w = nchunk * CHUNK
    pad = per_w - ew
    dst = edge_list[0].reshape(NW, ew)
    src = edge_list[1].reshape(NW, ew)
    srcp = jnp.concatenate(
        [src, jnp.full((NW, pad), n_in, jnp.int32)], axis=1)
    dstp = jnp.concatenate(
        [dst, jnp.zeros((NW, pad), jnp.int32)], axis=1)
    rows_per_w = per_w // SUB
    sidx_hbm = srcp.reshape(NW * rows_per_w, SUB)
    didx_hbm = dstp.reshape(NW * rows_per_w, SUB)

    acc_n = n_pad
    acc_per_tile = acc_n // NS
    zeros_hbm = jnp.zeros((acc_n, BCH), jnp.float32)

    mesh = plsc.VectorSubcoreMesh(core_axis_name="c", subcore_axis_name="s",
                                  num_cores=NC, num_subcores=NS)
    part = pl.kernel(
        functools.partial(_edge_body, nchunk, rows_per_w, acc_per_tile),
        out_type=jax.ShapeDtypeStruct((NC, acc_n, BCH), jnp.float32),
        mesh=mesh,
        compiler_params=pltpu.CompilerParams(needs_layout_passes=False,
                                             use_tc_tiling_on_sc=False),
        scratch_types=(
            [pltpu.VMEM((rows_per_w // 2, SUB), jnp.int32),
             pltpu.VMEM((rows_per_w // 2, SUB), jnp.int32)]
            + [pltpu.VMEM((CHUNK, 16), jnp.float32),
               pltpu.VMEM((CHUNK, 16), jnp.float32),
               pltpu.VMEM((CHUNK, BCH), jnp.float32)] * 2
            + [pltpu.VMEM_SHARED((acc_n, BCH), jnp.float32),
               pltpu.SemaphoreType.DMA, pltpu.SemaphoreType.DMA,
               pltpu.SemaphoreType.DMA, pltpu.SemaphoreType.DMA]),
    )(stab, dtab, sidx_hbm, didx_hbm, zeros_hbm)

    cm = 2176
    out_pad = pl.pallas_call(
        _merge_body,
        grid=(n_pad // cm,),
        in_specs=[pl.BlockSpec((NC, cm, BCH), lambda i: (0, i, 0))],
        out_specs=pl.BlockSpec((BCH, cm), lambda i: (0, i)),
        out_shape=jax.ShapeDtypeStruct((BCH, n_pad), jnp.float32),
    )(part)
    return out_pad[:, :n_out]


# speedup vs baseline: 1.1598x; 1.1598x over previous
import functools

import jax
import jax.numpy as jnp
from jax import lax
from jax.experimental import pallas as pl
from jax.experimental.pallas import tpu as pltpu
from jax.experimental.pallas import tpu_sc as plsc

NC = 2
NS = 16
NW = NC * NS
LANES = 16

BCH = 8
CHUNK = 256
SUB = 128
KSUB = CHUNK // SUB


def _prep_body(p_ref, q_ref, sig_ref, xat_ref, stab_ref, dtab_ref):
    p = p_ref[...]
    q = q_ref[...]
    sig = sig_ref[...]
    inv = 1.0 / (sig * sig)
    pn = jnp.sum(p * p, axis=1, keepdims=True)
    qn = jnp.sum(q * q, axis=1, keepdims=True)
    z1 = jnp.zeros_like(sig)
    z3 = jnp.zeros_like(p)
    stab_ref[...] = jnp.concatenate(
        [2.0 * inv * p, -pn * inv, -inv, z3, xat_ref[...]], axis=1)
    dtab_ref[...] = jnp.concatenate([q, z1, qn, z3], axis=1)


def _edge_body(ew, hslot, acc_per_tile, dump,
               stab, dtab, src_hbm, dst_hbm, out_hbm,
               sidx, didx, srows0, drows0, contrib0,
               srows1, drows1, contrib1,
               acc, gsem0, gsem1, ssem0, ssem1):
    c = lax.axis_index("c")
    s = lax.axis_index("s")
    wid = s * NC + c

    zvec = jnp.zeros((LANES,), jnp.float32)
    ziota = lax.iota(jnp.int32, LANES)
    zrows = ziota // BCH
    zcols = ziota % BCH

    def zbody(r, _):
        plsc.store_scatter(contrib0, [zrows + 2 * r, zcols], zvec)
        return 0

    lax.fori_loop(0, CHUNK * BCH // LANES, zbody, 0)
    abase = s * acc_per_tile
    nfull = acc_per_tile // CHUNK
    for r in range(nfull):
        pltpu.sync_copy(contrib0.at[pl.ds(0, CHUNK)],
                        acc.at[pl.ds(abase + r * CHUNK, CHUNK)])
    rem = acc_per_tile - nfull * CHUNK
    if rem:
        pltpu.sync_copy(contrib0.at[pl.ds(0, rem)],
                        acc.at[pl.ds(abase + nfull * CHUNK, rem)])
    plsc.subcore_barrier()

    bufs = ((srows0, drows0, contrib0, gsem0, ssem0),
            (srows1, drows1, contrib1, gsem1, ssem1))

    def goff(k, j):
        return pl.multiple_of(k * CHUNK + j * SUB, SUB)

    def fire_gathers(k, b):
        srows, drows, gsem = b[0], b[1], b[3]
        for j in range(KSUB):
            pltpu.async_copy(stab.at[sidx.at[pl.ds(goff(k, j), SUB)]],
                             srows.at[pl.ds(j * SUB, SUB)], gsem)
            pltpu.async_copy(dtab.at[didx.at[pl.ds(goff(k, j), SUB)]],
                             drows.at[pl.ds(j * SUB, SUB)], gsem)

    def drain_gathers(k, b):
        srows, drows, gsem = b[0], b[1], b[3]
        for j in range(KSUB):
            pltpu.make_async_copy(stab.at[sidx.at[pl.ds(goff(k, j), SUB)]],
                                  srows.at[pl.ds(j * SUB, SUB)], gsem).wait()
            pltpu.make_async_copy(dtab.at[didx.at[pl.ds(goff(k, j), SUB)]],
                                  drows.at[pl.ds(j * SUB, SUB)], gsem).wait()

    def compute(b):
        srows, drows, contrib = b[0], b[1], b[2]

        unroll = 2

        def cbody(i, _):
            prods = []
            for u in range(unroll):
                rows = lax.iota(jnp.int32, LANES) + (i * unroll + u) * LANES

                def gs(col, rows=rows):
                    return plsc.load_gather(
                        srows, [rows, jnp.full((LANES,), col, jnp.int32)])

                def gd(col, rows=rows):
                    return plsc.load_gather(
                        drows, [rows, jnp.full((LANES,), col, jnp.int32)])

                y = ((gs(0) * gd(0) + gs(1) * gd(1))
                     + (gs(2) * gd(2) + gs(3))
                     + gs(4) * gd(4))
                w = jnp.exp(y)
                prods.append((rows, [w * gs(8 + b_) for b_ in range(BCH)]))
            for rows, vals in prods:
                for b_ in range(BCH):
                    plsc.store_scatter(
                        contrib, [rows, jnp.full((LANES,), b_, jnp.int32)],
                        vals[b_])
            return 0

        lax.fori_loop(0, CHUNK // (LANES * unroll), cbody, 0)

    def fire_scatters(k, b):
        contrib, ssem = b[2], b[4]
        for j in range(KSUB):
            pltpu.async_copy(contrib.at[pl.ds(j * SUB, SUB)],
                             acc.at[didx.at[pl.ds(goff(k, j), SUB)]],
                             ssem, add=True)

    def drain_scatters(k, b):
        contrib, ssem = b[2], b[4]
        for j in range(KSUB):
            pltpu.make_async_copy(contrib.at[pl.ds(j * SUB, SUB)],
                                  acc.at[didx.at[pl.ds(goff(k, j), SUB)]],
                                  ssem).wait()

    nchunk_h = hslot // CHUNK

    def phase(t, k, this_b, other_b):
        @pl.when(k + 1 < nchunk_h)
        def _():
            fire_gathers(k + 1, other_b)
        drain_gathers(k, this_b)

        @pl.when(t >= 1)
        def _():
            drain_scatters(k, this_b)
        compute(this_b)
        fire_scatters(k, this_b)

    for h in range(2):
        start = wid * ew + h * hslot
        real_n = min(hslot, ew - h * hslot)
        pltpu.sync_copy(src_hbm.at[pl.ds(start, real_n)],
                        sidx.at[pl.ds(0, real_n)])
        pltpu.sync_copy(dst_hbm.at[pl.ds(start, real_n)],
                        didx.at[pl.ds(0, real_n)])
        for t_ in range((hslot - real_n) // LANES):
            off = real_n + t_ * LANES
            sidx[pl.ds(off, LANES)] = jnp.zeros((LANES,), jnp.int32)
            didx[pl.ds(off, LANES)] = jnp.full((LANES,), dump, jnp.int32)
        fire_gathers(0, bufs[0])

        def body(t, carry):
            phase(t, 2 * t, bufs[0], bufs[1])
            phase(t, 2 * t + 1, bufs[1], bufs[0])
            return carry

        lax.fori_loop(0, nchunk_h // 2, body, 0)
        drain_scatters(0, bufs[0])
        drain_scatters(0, bufs[1])

    plsc.subcore_barrier()
    pltpu.sync_copy(acc.at[pl.ds(s * acc_per_tile, acc_per_tile)],
                    out_hbm.at[c].at[pl.ds(s * acc_per_tile, acc_per_tile)])


def _merge_body(part_ref, out_ref):
    out_ref[...] = part_ref[0] + part_ref[1]


def kernel(x, inp_positions, out_positions, alpha, sigma, edge_list):
    n_in = inp_positions.shape[0]
    n_out = out_positions.shape[0]
    e = edge_list.shape[1]
    assert n_out == n_in

    xat = (alpha * x).T
    cn = 2000
    assert n_in % cn == 0
    grid = n_in // cn
    stab, dtab = pl.pallas_call(
        _prep_body,
        grid=(grid,),
        in_specs=[
            pl.BlockSpec((cn, 3), lambda i: (i, 0)),
            pl.BlockSpec((cn, 3), lambda i: (i, 0)),
            pl.BlockSpec((cn, 1), lambda i: (i, 0)),
            pl.BlockSpec((cn, BCH), lambda i: (i, 0)),
        ],
        out_specs=[
            pl.BlockSpec((cn, 16), lambda i: (i, 0)),
            pl.BlockSpec((cn, BCH), lambda i: (i, 0)),
        ],
        out_shape=[
            jax.ShapeDtypeStruct((n_in, 16), jnp.float32),
            jax.ShapeDtypeStruct((n_in, BCH), jnp.float32),
        ],
    )(inp_positions, out_positions, sigma.reshape(n_in, 1), xat)

    ew = e // NW
    assert ew * NW == e and ew % 8 == 0
    hslot = -(-ew // (2 * 2 * CHUNK)) * (2 * CHUNK)
    assert (hslot - min(hslot, ew - hslot)) % LANES == 0

    n_pad = -(-n_out // 128) * 128
    acc_per_tile = n_pad // NS
    dump = n_out

    src1d = edge_list[1]
    dst1d = edge_list[0]

    mesh = plsc.VectorSubcoreMesh(core_axis_name="c", subcore_axis_name="s",
                                  num_cores=NC, num_subcores=NS)
    part = pl.kernel(
        functools.partial(_edge_body, ew, hslot, acc_per_tile, dump),
        out_type=jax.ShapeDtypeStruct((NC, n_pad, BCH), jnp.float32),
        mesh=mesh,
        compiler_params=pltpu.CompilerParams(needs_layout_passes=False,
                                             use_tc_tiling_on_sc=False),
        scratch_types=(
            [pltpu.VMEM((hslot,), jnp.int32),
             pltpu.VMEM((hslot,), jnp.int32)]
            + [pltpu.VMEM((CHUNK, 16), jnp.float32),
               pltpu.VMEM((CHUNK, BCH), jnp.float32),
               pltpu.VMEM((CHUNK, BCH), jnp.float32)] * 2
            + [pltpu.VMEM_SHARED((n_pad, BCH), jnp.float32),
               pltpu.SemaphoreType.DMA, pltpu.SemaphoreType.DMA,
               pltpu.SemaphoreType.DMA, pltpu.SemaphoreType.DMA]),
    )(stab, dtab, src1d, dst1d)

    cm = 2000
    merged = pl.pallas_call(
        _merge_body,
        grid=(n_out // cm,),
        in_specs=[pl.BlockSpec((NC, cm, BCH), lambda i: (0, i, 0))],
        out_specs=pl.BlockSpec((cm, BCH), lambda i: (i, 0)),
        out_shape=jax.ShapeDtypeStruct((n_out, BCH), jnp.float32),
    )(part)
    return merged.T


# speedup vs baseline: 1.2110x; 1.0441x over previous
import functools

import jax
import jax.numpy as jnp
from jax import lax
from jax.experimental import pallas as pl
from jax.experimental.pallas import tpu as pltpu
from jax.experimental.pallas import tpu_sc as plsc

NC = 2
NS = 16
NW = NC * NS
LANES = 16

BCH = 8
CHUNK = 256
SUB = 128
KSUB = CHUNK // SUB


def _prep_body(p_ref, q_ref, sig_ref, stab_ref, dtab_ref):
    p = p_ref[...]
    q = q_ref[...]
    sig = sig_ref[...]
    inv = 1.0 / (sig * sig)
    pn = jnp.sum(p * p, axis=1, keepdims=True)
    qn = jnp.sum(q * q, axis=1, keepdims=True)
    z1 = jnp.zeros_like(sig)
    z3 = jnp.zeros_like(p)
    stab_ref[...] = jnp.concatenate([2.0 * inv * p, -pn * inv, -inv, z3],
                                    axis=1)
    dtab_ref[...] = jnp.concatenate([q, z1, qn, z3], axis=1)


def _edge_body(ew, hslot, acc_per_tile, dump,
               stab, xat, dtab, edge_hbm, out_hbm,
               sidx, didx, srows0, xrows0, drows0, contrib0,
               srows1, xrows1, drows1, contrib1,
               acc, gsem0, gsem1, ssem0, ssem1):
    c = lax.axis_index("c")
    s = lax.axis_index("s")
    wid = s * NC + c

    zvec = jnp.zeros((LANES,), jnp.float32)
    ziota = lax.iota(jnp.int32, LANES)
    zrows = ziota // BCH
    zcols = ziota % BCH

    def zbody(r, _):
        plsc.store_scatter(contrib0, [zrows + 2 * r, zcols], zvec)
        return 0

    lax.fori_loop(0, CHUNK * BCH // LANES, zbody, 0)
    abase = s * acc_per_tile
    nfull = acc_per_tile // CHUNK
    for r in range(nfull):
        pltpu.sync_copy(contrib0.at[pl.ds(0, CHUNK)],
                        acc.at[pl.ds(abase + r * CHUNK, CHUNK)])
    rem = acc_per_tile - nfull * CHUNK
    if rem:
        pltpu.sync_copy(contrib0.at[pl.ds(0, rem)],
                        acc.at[pl.ds(abase + nfull * CHUNK, rem)])
    plsc.subcore_barrier()

    bufs = ((srows0, xrows0, drows0, contrib0, gsem0, ssem0),
            (srows1, xrows1, drows1, contrib1, gsem1, ssem1))

    def goff(k, j):
        return pl.multiple_of(k * CHUNK + j * SUB, SUB)

    def fire_gathers(k, b):
        srows, xrows, drows, gsem = b[0], b[1], b[2], b[4]
        for j in range(KSUB):
            si = sidx.at[pl.ds(goff(k, j), SUB)]
            pltpu.async_copy(stab.at[si], srows.at[pl.ds(j * SUB, SUB)], gsem)
            pltpu.async_copy(xat.at[si], xrows.at[pl.ds(j * SUB, SUB)], gsem)
            pltpu.async_copy(dtab.at[didx.at[pl.ds(goff(k, j), SUB)]],
                             drows.at[pl.ds(j * SUB, SUB)], gsem)

    def drain_gathers(k, b):
        srows, xrows, drows, gsem = b[0], b[1], b[2], b[4]
        for j in range(KSUB):
            si = sidx.at[pl.ds(goff(k, j), SUB)]
            pltpu.make_async_copy(stab.at[si],
                                  srows.at[pl.ds(j * SUB, SUB)], gsem).wait()
            pltpu.make_async_copy(xat.at[si],
                                  xrows.at[pl.ds(j * SUB, SUB)], gsem).wait()
            pltpu.make_async_copy(dtab.at[didx.at[pl.ds(goff(k, j), SUB)]],
                                  drows.at[pl.ds(j * SUB, SUB)], gsem).wait()

    def compute(b):
        srows, xrows, drows, contrib = b[0], b[1], b[2], b[3]

        unroll = 2

        def cbody(i, _):
            prods = []
            for u in range(unroll):
                rows = lax.iota(jnp.int32, LANES) + (i * unroll + u) * LANES

                def gs(col, rows=rows):
                    return plsc.load_gather(
                        srows, [rows, jnp.full((LANES,), col, jnp.int32)])

                def gx(col, rows=rows):
                    return plsc.load_gather(
                        xrows, [rows, jnp.full((LANES,), col, jnp.int32)])

                def gd(col, rows=rows):
                    return plsc.load_gather(
                        drows, [rows, jnp.full((LANES,), col, jnp.int32)])

                y = ((gs(0) * gd(0) + gs(1) * gd(1))
                     + (gs(2) * gd(2) + gs(3))
                     + gs(4) * gd(4))
                w = jnp.exp(y)
                prods.append((rows, [w * gx(b_) for b_ in range(BCH)]))
            for rows, vals in prods:
                for b_ in range(BCH):
                    plsc.store_scatter(
                        contrib, [rows, jnp.full((LANES,), b_, jnp.int32)],
                        vals[b_])
            return 0

        lax.fori_loop(0, CHUNK // (LANES * unroll), cbody, 0)

    def fire_scatters(k, b):
        contrib, ssem = b[3], b[5]
        for j in range(KSUB):
            pltpu.async_copy(contrib.at[pl.ds(j * SUB, SUB)],
                             acc.at[didx.at[pl.ds(goff(k, j), SUB)]],
                             ssem, add=True)

    def drain_scatters(k, b):
        contrib, ssem = b[3], b[5]
        for j in range(KSUB):
            pltpu.make_async_copy(contrib.at[pl.ds(j * SUB, SUB)],
                                  acc.at[didx.at[pl.ds(goff(k, j), SUB)]],
                                  ssem).wait()

    nchunk_h = hslot // CHUNK

    def phase(t, k, this_b, other_b):
        @pl.when(k + 1 < nchunk_h)
        def _():
            fire_gathers(k + 1, other_b)
        drain_gathers(k, this_b)

        @pl.when(t >= 1)
        def _():
            drain_scatters(k, this_b)
        compute(this_b)
        fire_scatters(k, this_b)

    for h in range(2):
        start = wid * ew + h * hslot
        real_n = min(hslot, ew - h * hslot)
        pltpu.sync_copy(edge_hbm.at[1, pl.ds(start, real_n)],
                        sidx.at[pl.ds(0, real_n)])
        pltpu.sync_copy(edge_hbm.at[0, pl.ds(start, real_n)],
                        didx.at[pl.ds(0, real_n)])
        for t_ in range((hslot - real_n) // LANES):
            off = real_n + t_ * LANES
            sidx[pl.ds(off, LANES)] = jnp.zeros((LANES,), jnp.int32)
            didx[pl.ds(off, LANES)] = jnp.full((LANES,), dump, jnp.int32)
        fire_gathers(0, bufs[0])

        def body(t, carry):
            phase(t, 2 * t, bufs[0], bufs[1])
            phase(t, 2 * t + 1, bufs[1], bufs[0])
            return carry

        lax.fori_loop(0, nchunk_h // 2, body, 0)
        drain_scatters(0, bufs[0])
        drain_scatters(0, bufs[1])

    plsc.subcore_barrier()
    pltpu.sync_copy(acc.at[pl.ds(s * acc_per_tile, acc_per_tile)],
                    out_hbm.at[c].at[pl.ds(s * acc_per_tile, acc_per_tile)])


def _merge_body(part_ref, out_ref):
    out_ref[...] = part_ref[0] + part_ref[1]


def kernel(x, inp_positions, out_positions, alpha, sigma, edge_list):
    n_in = inp_positions.shape[0]
    n_out = out_positions.shape[0]
    e = edge_list.shape[1]
    assert n_out == n_in

    xat = (alpha * x).T
    cn = 2000
    assert n_in % cn == 0
    grid = n_in // cn
    stab, dtab = pl.pallas_call(
        _prep_body,
        grid=(grid,),
        in_specs=[
            pl.BlockSpec((cn, 3), lambda i: (i, 0)),
            pl.BlockSpec((cn, 3), lambda i: (i, 0)),
            pl.BlockSpec((cn, 1), lambda i: (i, 0)),
        ],
        out_specs=[
            pl.BlockSpec((cn, BCH), lambda i: (i, 0)),
            pl.BlockSpec((cn, BCH), lambda i: (i, 0)),
        ],
        out_shape=[
            jax.ShapeDtypeStruct((n_in, BCH), jnp.float32),
            jax.ShapeDtypeStruct((n_in, BCH), jnp.float32),
        ],
    )(inp_positions, out_positions, sigma.reshape(n_in, 1))

    ew = e // NW
    assert ew * NW == e and ew % 8 == 0
    hslot = -(-ew // (2 * 2 * CHUNK)) * (2 * CHUNK)
    assert (hslot - min(hslot, ew - hslot)) % LANES == 0

    n_pad = -(-n_out // 128) * 128
    acc_per_tile = n_pad // NS
    dump = n_out

    mesh = plsc.VectorSubcoreMesh(core_axis_name="c", subcore_axis_name="s",
                                  num_cores=NC, num_subcores=NS)
    part = pl.kernel(
        functools.partial(_edge_body, ew, hslot, acc_per_tile, dump),
        out_type=jax.ShapeDtypeStruct((NC, n_pad, BCH), jnp.float32),
        mesh=mesh,
        compiler_params=pltpu.CompilerParams(needs_layout_passes=False,
                                             use_tc_tiling_on_sc=False),
        scratch_types=(
            [pltpu.VMEM((hslot,), jnp.int32),
             pltpu.VMEM((hslot,), jnp.int32)]
            + [pltpu.VMEM((CHUNK, BCH), jnp.float32)] * 8
            + [pltpu.VMEM_SHARED((n_pad, BCH), jnp.float32),
               pltpu.SemaphoreType.DMA, pltpu.SemaphoreType.DMA,
               pltpu.SemaphoreType.DMA, pltpu.SemaphoreType.DMA]),
    )(stab, xat, dtab, edge_list)

    cm = 2000
    merged = pl.pallas_call(
        _merge_body,
        grid=(n_out // cm,),
        in_specs=[pl.BlockSpec((NC, cm, BCH), lambda i: (0, i, 0))],
        out_specs=pl.BlockSpec((cm, BCH), lambda i: (i, 0)),
        out_shape=jax.ShapeDtypeStruct((n_out, BCH), jnp.float32),
    )(part)
    return merged.T
